# baseline (speedup 1.0000x reference)
import functools

import jax
import jax.numpy as jnp
from jax import lax
from jax.experimental import pallas as pl
from jax.experimental.pallas import tpu as pltpu
from jax.experimental.pallas import tpu_sc as plsc

DIMK = 256
NCODES = 8192
NPIX = 16384
BI = 256
NBLK = NPIX // BI
CJ = 1024
NJ = NCODES // CJ

SC_NC = 2
SC_NS = 16
SC_NW = SC_NC * SC_NS
B_PER_W = NPIX // SC_NW
SC_CHUNK = 256
SC_NCHUNK = B_PER_W // SC_CHUNK


def _tc_body(flat_ref, embed_ref, idx_ref, loss_ref):
    f = flat_ref[...]
    f2 = jnp.sum(f * f, axis=1, keepdims=True)
    best = jnp.full((BI, 1), jnp.inf, dtype=jnp.float32)
    bidx = jnp.zeros((BI, 1), dtype=jnp.int32)
    for jc in range(NJ):
        e_blk = embed_ref[:, jc * CJ:(jc + 1) * CJ]
        e2 = jnp.sum(e_blk * e_blk, axis=0, keepdims=True)
        s = jnp.dot(f, e_blk, preferred_element_type=jnp.float32,
                    precision=lax.Precision.HIGHEST)
        dist = (f2 - 2.0 * s) + e2
        cmin = jnp.min(dist, axis=1, keepdims=True)
        ids = lax.broadcasted_iota(jnp.int32, (BI, CJ), 1) + jc * CJ
        cidx = jnp.min(jnp.where(dist == cmin, ids, NCODES), axis=1,
                       keepdims=True)
        upd = cmin < best
        best = jnp.where(upd, cmin, best)
        bidx = jnp.where(upd, cidx, bidx)
    idx_ref[0] = bidx
    part = jnp.sum(best, axis=0, keepdims=True)
    i = pl.program_id(0)

    @pl.when(i == 0)
    def _init():
        loss_ref[...] = part

    @pl.when(i > 0)
    def _acc():
        loss_ref[...] += part


def _tc_assign():
    return pl.pallas_call(
        _tc_body,
        grid=(NBLK,),
        in_specs=[
            pl.BlockSpec((BI, DIMK), lambda i: (i, 0)),
            pl.BlockSpec((DIMK, NCODES), lambda i: (0, 0)),
        ],
        out_specs=[
            pl.BlockSpec((1, BI, 1), lambda i: (i, 0, 0)),
            pl.BlockSpec((1, 1), lambda i: (0, 0)),
        ],
        out_shape=[
            jax.ShapeDtypeStruct((NBLK, BI, 1), jnp.int32),
            jax.ShapeDtypeStruct((1, 1), jnp.float32),
        ],
    )


def _sc_gather_body(table_hbm, idx_hbm, out_hbm, idx_v, rows_v, sem):
    wid = lax.axis_index("s") * SC_NC + lax.axis_index("c")
    base = wid * B_PER_W
    pltpu.sync_copy(idx_hbm.at[pl.ds(base, B_PER_W)], idx_v)
    for c in range(SC_NCHUNK):
        pltpu.async_copy(
            table_hbm.at[idx_v.at[pl.ds(c * SC_CHUNK, SC_CHUNK)]],
            rows_v, sem).wait()
        pltpu.sync_copy(rows_v, out_hbm.at[pl.ds(base + c * SC_CHUNK, SC_CHUNK)])


@functools.lru_cache(maxsize=None)
def _sc_gather():
    return functools.partial(
        pl.kernel,
        mesh=plsc.VectorSubcoreMesh(core_axis_name="c", subcore_axis_name="s"),
        out_type=jax.ShapeDtypeStruct((NPIX, DIMK), jnp.float32),
        scratch_types=[
            pltpu.VMEM((B_PER_W,), jnp.int32),
            pltpu.VMEM((SC_CHUNK, DIMK), jnp.float32),
            pltpu.SemaphoreType.DMA,
        ],
    )(_sc_gather_body)


def kernel(x, embed):
    x_p = jnp.transpose(x, (0, 2, 3, 1)).reshape(NPIX, DIMK)
    idx3, loss_acc = _tc_assign()(x_p, embed)
    idx = idx3.reshape(NPIX)
    table = embed.T
    quant = _sc_gather()(table, idx)
    out = jnp.transpose(quant.reshape(16, 32, 32, DIMK), (0, 3, 1, 2))
    quant_loss = loss_acc[0, 0] / (NPIX * DIMK)
    return out, quant_loss


# speedup vs baseline: 1.7475x; 1.7475x over previous
import functools

import jax
import jax.numpy as jnp
from jax import lax
from jax.experimental import pallas as pl
from jax.experimental.pallas import tpu as pltpu
from jax.experimental.pallas import tpu_sc as plsc

DIMK = 256
NCODES = 8192
NPIX = 16384
BI = 256
NBLK = NPIX // BI
CJ = 1024
NJ = NCODES // CJ

SC_NC = 2
SC_NS = 16
SC_NW = SC_NC * SC_NS
B_PER_W = NPIX // SC_NW
SC_CHUNK = 256
SC_NCHUNK = B_PER_W // SC_CHUNK


def _tc_body(flat_ref, embed_ref, idx_ref, loss_ref):
    f = flat_ref[...]
    f2 = jnp.sum(f * f, axis=1, keepdims=True)
    best = jnp.full((BI, 1), jnp.inf, dtype=jnp.float32)
    bidx = jnp.zeros((BI, 1), dtype=jnp.int32)
    for jc in range(NJ):
        e_blk = embed_ref[:, jc * CJ:(jc + 1) * CJ]
        e2 = jnp.sum(e_blk * e_blk, axis=0, keepdims=True)
        s = jnp.dot(f, e_blk, preferred_element_type=jnp.float32)
        dist = (f2 - 2.0 * s) + e2
        cmin = jnp.min(dist, axis=1, keepdims=True)
        ids = lax.broadcasted_iota(jnp.int32, (BI, CJ), 1) + jc * CJ
        cidx = jnp.min(jnp.where(dist == cmin, ids, NCODES), axis=1,
                       keepdims=True)
        upd = cmin < best
        best = jnp.where(upd, cmin, best)
        bidx = jnp.where(upd, cidx, bidx)
    idx_ref[0] = bidx
    part = jnp.sum(best, axis=0, keepdims=True)
    i = pl.program_id(0)

    @pl.when(i == 0)
    def _init():
        loss_ref[...] = part

    @pl.when(i > 0)
    def _acc():
        loss_ref[...] += part


def _tc_assign():
    return pl.pallas_call(
        _tc_body,
        grid=(NBLK,),
        in_specs=[
            pl.BlockSpec((BI, DIMK), lambda i: (i, 0)),
            pl.BlockSpec((DIMK, NCODES), lambda i: (0, 0)),
        ],
        out_specs=[
            pl.BlockSpec((1, BI, 1), lambda i: (i, 0, 0)),
            pl.BlockSpec((1, 1), lambda i: (0, 0)),
        ],
        out_shape=[
            jax.ShapeDtypeStruct((NBLK, BI, 1), jnp.int32),
            jax.ShapeDtypeStruct((1, 1), jnp.float32),
        ],
    )


def _sc_gather_body(table_hbm, idx_hbm, out_hbm, idx_v, rows_v, sem):
    wid = lax.axis_index("s") * SC_NC + lax.axis_index("c")
    base = wid * B_PER_W
    pltpu.sync_copy(idx_hbm.at[pl.ds(base, B_PER_W)], idx_v)
    for c in range(SC_NCHUNK):
        pltpu.async_copy(
            table_hbm.at[idx_v.at[pl.ds(c * SC_CHUNK, SC_CHUNK)]],
            rows_v, sem).wait()
        pltpu.sync_copy(rows_v, out_hbm.at[pl.ds(base + c * SC_CHUNK, SC_CHUNK)])


@functools.lru_cache(maxsize=None)
def _sc_gather():
    return functools.partial(
        pl.kernel,
        mesh=plsc.VectorSubcoreMesh(core_axis_name="c", subcore_axis_name="s"),
        out_type=jax.ShapeDtypeStruct((NPIX, DIMK), jnp.float32),
        scratch_types=[
            pltpu.VMEM((B_PER_W,), jnp.int32),
            pltpu.VMEM((SC_CHUNK, DIMK), jnp.float32),
            pltpu.SemaphoreType.DMA,
        ],
    )(_sc_gather_body)


def kernel(x, embed):
    x_p = jnp.transpose(x, (0, 2, 3, 1)).reshape(NPIX, DIMK)
    idx3, loss_acc = _tc_assign()(x_p, embed)
    idx = idx3.reshape(NPIX)
    table = embed.T
    quant = _sc_gather()(table, idx)
    out = jnp.transpose(quant.reshape(16, 32, 32, DIMK), (0, 3, 1, 2))
    quant_loss = loss_acc[0, 0] / (NPIX * DIMK)
    return out, quant_loss
